# Initial kernel scaffold; baseline (speedup 1.0000x reference)
import jax
import jax.numpy as jnp
from jax.experimental import pallas as pl


def kernel(x, adj_norm, W, b):
    raise NotImplementedError("write your pallas kernel here")


# speedup vs baseline: 1.0520x; 1.0520x over previous
import jax
import jax.numpy as jnp
from jax.experimental import pallas as pl
from jax.experimental.pallas import tpu as pltpu

_BM = 400


def _hop1_kernel(a_ref, x_ref, y_ref):
    y_ref[...] = jax.lax.dot_general(
        a_ref[...], x_ref[...], (((1,), (0,)), ((), ())),
        preferred_element_type=jnp.float32)


def _hop2_kernel(a_ref, y_ref, w_ref, b_ref, o_ref):
    h = jax.lax.dot_general(
        a_ref[...], y_ref[...], (((1,), (0,)), ((), ())),
        preferred_element_type=jnp.float32)
    logits = jax.lax.dot_general(
        h, w_ref[...], (((1,), (1,)), ((), ())),
        preferred_element_type=jnp.float32)
    logits = logits + b_ref[...]
    m = jnp.max(logits, axis=1, keepdims=True)
    shifted = logits - m
    lse = jnp.log(jnp.sum(jnp.exp(shifted), axis=1, keepdims=True))
    o_ref[...] = shifted - lse


def kernel(x, adj_norm, W, b):
    n, nfeat = x.shape
    nclass = W.shape[0]
    grid = (n // _BM,)
    params = pltpu.CompilerParams(vmem_limit_bytes=100 * 2**20)

    y = pl.pallas_call(
        _hop1_kernel,
        grid=grid,
        in_specs=[
            pl.BlockSpec((_BM, n), lambda i: (i, 0)),
            pl.BlockSpec((n, nfeat), lambda i: (0, 0)),
        ],
        out_specs=pl.BlockSpec((_BM, nfeat), lambda i: (i, 0)),
        out_shape=jax.ShapeDtypeStruct((n, nfeat), jnp.float32),
        compiler_params=params,
    )(adj_norm, x)

    out = pl.pallas_call(
        _hop2_kernel,
        grid=grid,
        in_specs=[
            pl.BlockSpec((_BM, n), lambda i: (i, 0)),
            pl.BlockSpec((n, nfeat), lambda i: (0, 0)),
            pl.BlockSpec((nclass, nfeat), lambda i: (0, 0)),
            pl.BlockSpec((1, nclass), lambda i: (0, 0)),
        ],
        out_specs=pl.BlockSpec((_BM, nclass), lambda i: (i, 0)),
        out_shape=jax.ShapeDtypeStruct((n, nclass), jnp.float32),
        compiler_params=params,
    )(adj_norm, y, W, b.reshape(1, nclass))

    return out
